# baseline (speedup 1.0000x reference)
import numpy as np

import jax
import jax.numpy as jnp
from jax import lax
from jax.experimental import pallas as pl
from jax.experimental.pallas import tpu as pltpu
from jax.experimental.pallas import tpu_sc as plsc

_MAXC = 4
_LANES = 16
_NWORKERS = 32

def _sc_sample(weights, thresh, x):
    n_in, n_out = weights.shape
    bs = x.shape[0]
    nq = thresh.shape[1]
    rows_per_w = n_in // _NWORKERS
    n_chunk = n_out // _LANES
    n_grp = n_chunk // _LANES

    mesh = plsc.VectorSubcoreMesh(core_axis_name="c", subcore_axis_name="s")

    @jax.jit
    def run(w, t, m):
        @pl.kernel(
            mesh=mesh,
            compiler_params=pltpu.CompilerParams(
                needs_layout_passes=False, use_tc_tiling_on_sc=False),
            out_type=jax.ShapeDtypeStruct((bs, n_in, _MAXC), jnp.int32),
            scratch_types=[
                pltpu.VMEM((n_out,), jnp.float32),
                pltpu.VMEM((n_out,), jnp.float32),
                pltpu.VMEM((n_chunk,), jnp.float32),
                pltpu.VMEM((_LANES,), jnp.float32),
                pltpu.VMEM((rows_per_w, nq), jnp.float32),
                pltpu.VMEM((bs * rows_per_w,), jnp.int32),
                pltpu.VMEM((bs, rows_per_w, _MAXC), jnp.int32),
                pltpu.SemaphoreType.DMA,
                pltpu.SemaphoreType.DMA,
            ],
        )
        def k(w_hbm, t_hbm, x_hbm, out_hbm, row_a, row_b, l1_v, gc_v,
              t_v, x_v, o_v, sem_a, sem_b):
            wid = lax.axis_index("s") * 2 + lax.axis_index("c")
            base = wid * rows_per_w
            pltpu.sync_copy(t_hbm.at[pl.ds(base, rows_per_w)], t_v)
            for b in range(bs):
                pltpu.sync_copy(x_hbm.at[b, pl.ds(base, rows_per_w)],
                                x_v.at[pl.ds(b * rows_per_w, rows_per_w)])

            iota = lax.iota(jnp.int32, _LANES)
            last = rows_per_w - 1
            base_idx = iota * _LANES
            lane_b = iota // 4
            lane_k = iota % 4

            def chunk_sums(off):
                i0 = base_idx + off
                a0 = plsc.load_gather(row_v_cur[0], [i0])
                a1 = plsc.load_gather(row_v_cur[0], [i0 + 1])
                a2 = plsc.load_gather(row_v_cur[0], [i0 + 2])
                a3 = plsc.load_gather(row_v_cur[0], [i0 + 3])
                for j in range(4, _LANES, 4):
                    a0 = a0 + plsc.load_gather(row_v_cur[0], [i0 + j])
                    a1 = a1 + plsc.load_gather(row_v_cur[0], [i0 + j + 1])
                    a2 = a2 + plsc.load_gather(row_v_cur[0], [i0 + j + 2])
                    a3 = a3 + plsc.load_gather(row_v_cur[0], [i0 + j + 3])
                return (a0 + a1) + (a2 + a3)

            row_v_cur = [None]

            def cdf_build(row_v):
                row_v_cur[0] = row_v

                def grp_body(h, tot):
                    g = 2 * h
                    acc0 = chunk_sums(g * (_LANES * _LANES))
                    acc1 = chunk_sums((g + 1) * (_LANES * _LANES))
                    r0 = plsc.cumsum(acc0)
                    l1_v[pl.ds(g * _LANES, _LANES)] = r0
                    tot = jnp.where(iota == g, r0[_LANES - 1], tot)
                    r1 = plsc.cumsum(acc1)
                    l1_v[pl.ds((g + 1) * _LANES, _LANES)] = r1
                    tot = jnp.where(iota == g + 1, r1[_LANES - 1], tot)
                    return tot

                tot = lax.fori_loop(0, n_grp // 2, grp_body,
                                    jnp.zeros((_LANES,), jnp.float32))
                gc = plsc.cumsum(tot)
                gc_v[...] = gc
                return gc[_LANES - 1]

            def queries(row_v, s_tot, r):
                rsplat = jnp.full((_LANES,), 0, jnp.int32) + r
                for q in range(nq // _LANES):
                    u = t_v[r, pl.ds(q * _LANES, _LANES)]
                    xval = plsc.load_gather(
                        x_v, [(lane_b + q * 4) * rows_per_w + rsplat])
                    tt = u * s_tot

                    p0 = jnp.zeros((_LANES,), jnp.int32)
                    for step in (8, 4, 2, 1):
                        nxt = p0 + step
                        v = plsc.load_gather(gc_v, [nxt - 1])
                        p0 = jnp.where(v < tt, nxt, p0)
                    g0 = plsc.load_gather(gc_v, [jnp.maximum(p0 - 1, 0)])
                    base0 = jnp.where(p0 > 0, g0, jnp.float32(0.0))

                    cbase = p0 * _LANES
                    p1 = jnp.zeros((_LANES,), jnp.int32)
                    for step in (8, 4, 2, 1):
                        nxt = p1 + step
                        v = plsc.load_gather(l1_v, [cbase + nxt - 1])
                        p1 = jnp.where(base0 + v < tt, nxt, p1)
                    c0 = plsc.load_gather(l1_v, [cbase + jnp.maximum(p1 - 1, 0)])
                    runv = base0 + jnp.where(p1 > 0, c0, jnp.float32(0.0))

                    ebase = (cbase + p1) * _LANES
                    cnt = cbase + p1
                    cnt = cnt * _LANES
                    for j in range(_LANES):
                        wj = plsc.load_gather(row_v, [ebase + j])
                        runv = runv + wj
                        cnt = cnt + jnp.where(runv < tt, 1, 0)

                    rank = jnp.minimum(cnt, n_out - 1)
                    val = jnp.where(lane_k < xval, rank, -1)
                    plsc.store_scatter(o_v, [lane_b + q * 4, rsplat, lane_k], val)

            pltpu.async_copy(w_hbm.at[base], row_a, sem_a)

            def row_pair(i, carry):
                r0 = 2 * i
                r1 = jnp.minimum(r0 + 1, last)
                pltpu.async_copy(w_hbm.at[base + r1], row_b, sem_b)
                pltpu.make_async_copy(w_hbm.at[base], row_a, sem_a).wait()
                sa = cdf_build(row_a)
                queries(row_a, sa, r0)
                r2 = jnp.minimum(r0 + 2, last)
                pltpu.async_copy(w_hbm.at[base + r2], row_a, sem_a)
                pltpu.make_async_copy(w_hbm.at[base], row_b, sem_b).wait()
                sb = cdf_build(row_b)
                queries(row_b, sb, r1)
                return carry

            lax.fori_loop(0, rows_per_w // 2, row_pair, 0)
            pltpu.make_async_copy(w_hbm.at[base], row_a, sem_a).wait()
            for b in range(bs):
                pltpu.sync_copy(o_v.at[b], out_hbm.at[b, pl.ds(base, rows_per_w), :])

        return k(w, t, m)

    return run(weights, thresh, x)


def kernel(x, weights):
    bs, n_in = x.shape
    u = jax.random.uniform(jax.random.key(42), (bs, n_in, _MAXC), dtype=jnp.float32)
    thresh = u.transpose(1, 0, 2).reshape(n_in, bs * _MAXC)
    return _sc_sample(weights, thresh, x)


# speedup vs baseline: 1.4048x; 1.4048x over previous
import jax
import jax.numpy as jnp
from jax import lax
from jax.experimental import pallas as pl
from jax.experimental.pallas import tpu as pltpu
from jax.experimental.pallas import tpu_sc as plsc

_MAXC = 4
_LANES = 16
_NWORKERS = 32


def _sc_sample(weights, thresh, mask):
    n_in, n_out = weights.shape
    nq = thresh.shape[1]
    rows_per_w = n_in // _NWORKERS
    n_chunk = n_out // _LANES
    n_grp = n_chunk // _LANES

    mesh = plsc.VectorSubcoreMesh(core_axis_name="c", subcore_axis_name="s")

    @jax.jit
    def run(w, t, m):
        @pl.kernel(
            mesh=mesh,
            compiler_params=pltpu.CompilerParams(needs_layout_passes=False),
            out_type=jax.ShapeDtypeStruct((n_in, nq), jnp.int32),
            scratch_types=[
                pltpu.VMEM((n_out,), jnp.float32),
                pltpu.VMEM((n_out,), jnp.float32),
                pltpu.VMEM((n_chunk,), jnp.float32),
                pltpu.VMEM((rows_per_w, nq), jnp.float32),
                pltpu.VMEM((rows_per_w, nq), jnp.int32),
                pltpu.VMEM((rows_per_w, nq), jnp.int32),
                pltpu.SemaphoreType.DMA,
                pltpu.SemaphoreType.DMA,
            ],
        )
        def k(w_hbm, t_hbm, m_hbm, out_hbm, row_a, row_b, l1_v, t_v, m_v, o_v,
              sem_a, sem_b):
            wid = lax.axis_index("s") * 2 + lax.axis_index("c")
            base = wid * rows_per_w
            pltpu.sync_copy(t_hbm.at[pl.ds(base, rows_per_w)], t_v)
            pltpu.sync_copy(m_hbm.at[pl.ds(base, rows_per_w)], m_v)

            iota = lax.iota(jnp.int32, _LANES)
            last = rows_per_w - 1
            base_idx = iota * _LANES

            def process(row_v, r):
                def chunk_sums(off):
                    i0 = base_idx + off
                    i1 = i0 + 1
                    a0 = plsc.load_gather(row_v, [i0])
                    a1 = plsc.load_gather(row_v, [i1])
                    for _ in range(_LANES // 2 - 1):
                        i0 = i0 + 2
                        i1 = i1 + 2
                        a0 = a0 + plsc.load_gather(row_v, [i0])
                        a1 = a1 + plsc.load_gather(row_v, [i1])
                    return a0 + a1

                def grp_body(h, carry):
                    g = 2 * h
                    acc0 = chunk_sums(g * (_LANES * _LANES))
                    acc1 = chunk_sums((g + 1) * (_LANES * _LANES))
                    sc0 = plsc.cumsum(acc0) + carry
                    l1_v[pl.ds(g * _LANES, _LANES)] = sc0
                    sc1 = plsc.cumsum(acc1) + sc0[_LANES - 1]
                    l1_v[pl.ds((g + 1) * _LANES, _LANES)] = sc1
                    return sc1[_LANES - 1]

                s_tot = lax.fori_loop(0, n_grp // 2, grp_body, jnp.float32(0.0))

                for q in range(nq // _LANES):
                    u = t_v[r, pl.ds(q * _LANES, _LANES)]
                    mv = m_v[r, pl.ds(q * _LANES, _LANES)]
                    tt = u * s_tot

                    pos = jnp.zeros((_LANES,), jnp.int32)
                    for step in (128, 64, 32, 16, 8, 4, 2, 1):
                        nxt = pos + step
                        v = plsc.load_gather(l1_v, [nxt - 1])
                        pos = jnp.where(v < tt, nxt, pos)

                    bidx = jnp.maximum(pos - 1, 0)
                    cdf0 = plsc.load_gather(l1_v, [bidx])
                    runv = jnp.where(pos > 0, cdf0, jnp.float32(0.0))

                    ebase = pos * _LANES
                    cnt = ebase
                    for j in range(_LANES):
                        wj = plsc.load_gather(row_v, [ebase + j])
                        runv = runv + wj
                        cnt = cnt + jnp.where(runv < tt, 1, 0)

                    rank = jnp.minimum(cnt, n_out - 1)
                    o_v[r, pl.ds(q * _LANES, _LANES)] = jnp.where(mv > 0, rank, -1)

            pltpu.async_copy(w_hbm.at[base], row_a, sem_a)

            def row_pair(i, carry):
                r0 = 2 * i
                r1 = jnp.minimum(r0 + 1, last)
                pltpu.async_copy(w_hbm.at[base + r1], row_b, sem_b)
                pltpu.make_async_copy(w_hbm.at[base], row_a, sem_a).wait()
                process(row_a, r0)
                r2 = jnp.minimum(r0 + 2, last)
                pltpu.async_copy(w_hbm.at[base + r2], row_a, sem_a)
                pltpu.make_async_copy(w_hbm.at[base], row_b, sem_b).wait()
                process(row_b, r1)
                return carry

            lax.fori_loop(0, rows_per_w // 2, row_pair, 0)
            pltpu.make_async_copy(w_hbm.at[base], row_a, sem_a).wait()
            pltpu.sync_copy(o_v, out_hbm.at[pl.ds(base, rows_per_w)])

        return k(w, t, m)

    return run(weights, thresh, mask)


def kernel(x, weights):
    bs, n_in = x.shape
    u = jax.random.uniform(jax.random.key(42), (bs, n_in, _MAXC), dtype=jnp.float32)
    thresh = u.transpose(1, 0, 2).reshape(n_in, bs * _MAXC)
    mask = (jnp.arange(_MAXC, dtype=jnp.int32)[None, None, :] < x[:, :, None])
    mask = mask.astype(jnp.int32).transpose(1, 0, 2).reshape(n_in, bs * _MAXC)
    out = _sc_sample(weights, thresh, mask)
    return out.reshape(n_in, bs, _MAXC).transpose(1, 0, 2)
